# baseline (speedup 1.0000x reference)
import functools

import jax
import jax.numpy as jnp
from jax import lax
from jax.experimental import pallas as pl
from jax.experimental.pallas import tpu as pltpu
from jax.experimental.pallas import tpu_sc as plsc

B = 16
C = 64
H = 11
W = 20
P = 2784
HW = H * W
NC = 2
NS = 16
NW = NC * NS
LANES = 16
NCH = P // LANES
FTOT = 2 * B * HW
SENTINEL = 1 << 20


HGROUPS = ((0, 8), (8, 3))


def _sc_body(feats_hbm, scode_hbm, out_hbm, scode_v, fbuf, obuf,
             sem0, sem1, sem2, sem3):
    cid = lax.axis_index("c")
    sid = lax.axis_index("s")
    wid = sid * NC + cid
    c0 = wid * 2

    pltpu.sync_copy(scode_hbm, scode_v)
    pltpu.sync_copy(feats_hbm.at[pl.ds(c0 * B * HW, FTOT)],
                    fbuf.at[pl.ds(0, FTOT)])
    fbuf[pl.ds(FTOT, LANES)] = jnp.zeros((LANES,), jnp.float32)
    sems = ((sem0, sem1), (sem2, sem3))

    def batch_body(b, _):
        for g, (h0, hn) in enumerate(HGROUPS):
            for ci in range(2):
                @pl.when(b > 0)
                def _drain():
                    pltpu.make_async_copy(
                        out_hbm.at[b, c0 + ci, pl.ds(h0, hn)],
                        obuf.at[ci, pl.ds(h0, hn)], sems[g][ci]).wait()

            for h in range(h0, h0 + hn):
                @plsc.parallel_loop(0, NCH, 1, unroll=4)
                def chunk_body(i):
                    off = i * LANES
                    xvec = scode_v[pl.ds(h * P + off, LANES)]
                    for ci in range(2):
                        iv = jnp.minimum(
                            xvec + (b * HW + ci * B * HW + h * W), FTOT)
                        obuf[ci, h, 0, pl.ds(off, LANES)] = plsc.load_gather(
                            fbuf, [iv])

            for ci in range(2):
                pltpu.make_async_copy(
                    obuf.at[ci, pl.ds(h0, hn)],
                    out_hbm.at[b, c0 + ci, pl.ds(h0, hn)],
                    sems[g][ci]).start()
        return 0

    lax.fori_loop(0, B, batch_body, 0)
    for g, (h0, hn) in enumerate(HGROUPS):
        for ci in range(2):
            pltpu.make_async_copy(out_hbm.at[B - 1, c0 + ci, pl.ds(h0, hn)],
                                  obuf.at[ci, pl.ds(h0, hn)],
                                  sems[g][ci]).wait()


_sc_gather = functools.partial(
    pl.kernel,
    out_type=jax.ShapeDtypeStruct((B, C, H, 1, P), jnp.float32),
    mesh=plsc.VectorSubcoreMesh(core_axis_name="c", subcore_axis_name="s",
                                num_cores=NC, num_subcores=NS),
    compiler_params=pltpu.CompilerParams(needs_layout_passes=False,
                                         disable_bounds_checks=True),
    scratch_types=[
        pltpu.VMEM((H * P,), jnp.int32),
        pltpu.VMEM((FTOT + LANES,), jnp.float32),
        pltpu.VMEM((2, H, 1, P), jnp.float32),
        pltpu.SemaphoreType.DMA,
        pltpu.SemaphoreType.DMA,
        pltpu.SemaphoreType.DMA,
        pltpu.SemaphoreType.DMA,
    ],
)(_sc_body)


def kernel(features, cut_zs, cut_ys, cut_xs, invalid_mask):
    del cut_zs, cut_ys
    stride = C * H
    cols = [lax.slice(cut_xs, (h,), (h + (P - 1) * stride + 1,), (stride,))
            for h in range(H)]
    xs_t = jnp.stack(cols, axis=0)
    inv_t = invalid_mask[:, 0, :, 0].T
    scode_t = jnp.where(inv_t, jnp.int32(SENTINEL), xs_t)
    feats_flat = features.transpose(1, 0, 2, 3).reshape(-1)
    out = _sc_gather(feats_flat, scode_t.reshape(-1))
    return out.transpose(0, 4, 1, 2, 3)


# speedup vs baseline: 1.2324x; 1.2324x over previous
import functools

import jax
import jax.numpy as jnp
from jax import lax
from jax.experimental import pallas as pl
from jax.experimental.pallas import tpu as pltpu
from jax.experimental.pallas import tpu_sc as plsc

B = 16
C = 64
H = 11
W = 20
P = 2784
HW = H * W
NC = 2
NS = 16
NW = NC * NS
LANES = 16
NCH = P // LANES
FTOT = 2 * B * HW
SENTINEL = 1 << 20


HGROUPS = ((0, 8), (8, 3))


def _sc_body(feats_hbm, scode_hbm, out_hbm, scode_v, fbuf, obuf,
             sem0, sem1, sem2, sem3):
    cid = lax.axis_index("c")
    sid = lax.axis_index("s")
    wid = sid * NC + cid
    c0 = wid * 2

    pltpu.sync_copy(scode_hbm, scode_v)
    pltpu.sync_copy(feats_hbm.at[pl.ds(c0 * B * HW, FTOT)],
                    fbuf.at[pl.ds(0, FTOT)])
    fbuf[pl.ds(FTOT, LANES)] = jnp.zeros((LANES,), jnp.float32)
    sems = ((sem0, sem1), (sem2, sem3))

    def batch_body(b, _):
        for g, (h0, hn) in enumerate(HGROUPS):
            for ci in range(2):
                @pl.when(b > 0)
                def _drain():
                    pltpu.make_async_copy(
                        out_hbm.at[b, c0 + ci, pl.ds(h0, hn)],
                        obuf.at[ci, pl.ds(h0, hn)], sems[g][ci]).wait()

            for h in range(h0, h0 + hn):
                @plsc.parallel_loop(0, NCH, 1, unroll=4)
                def chunk_body(i):
                    off = i * LANES
                    xvec = scode_v[pl.ds(h * P + off, LANES)]
                    for ci in range(2):
                        iv = jnp.minimum(
                            xvec + (b * HW + ci * B * HW + h * W), FTOT)
                        obuf[ci, h, 0, pl.ds(off, LANES)] = plsc.load_gather(
                            fbuf, [iv])

            for ci in range(2):
                pltpu.make_async_copy(
                    obuf.at[ci, pl.ds(h0, hn)],
                    out_hbm.at[b, c0 + ci, pl.ds(h0, hn)],
                    sems[g][ci]).start()
        return 0

    lax.fori_loop(0, B, batch_body, 0)
    for g, (h0, hn) in enumerate(HGROUPS):
        for ci in range(2):
            pltpu.make_async_copy(out_hbm.at[B - 1, c0 + ci, pl.ds(h0, hn)],
                                  obuf.at[ci, pl.ds(h0, hn)],
                                  sems[g][ci]).wait()


_sc_gather = functools.partial(
    pl.kernel,
    out_type=jax.ShapeDtypeStruct((B, C, H, 1, P), jnp.float32),
    mesh=plsc.VectorSubcoreMesh(core_axis_name="c", subcore_axis_name="s",
                                num_cores=NC, num_subcores=NS),
    compiler_params=pltpu.CompilerParams(needs_layout_passes=False,
                                         disable_bounds_checks=True),
    scratch_types=[
        pltpu.VMEM((H * P,), jnp.int32),
        pltpu.VMEM((FTOT + LANES,), jnp.float32),
        pltpu.VMEM((2, H, 1, P), jnp.float32),
        pltpu.SemaphoreType.DMA,
        pltpu.SemaphoreType.DMA,
        pltpu.SemaphoreType.DMA,
        pltpu.SemaphoreType.DMA,
    ],
)(_sc_body)


def kernel(features, cut_zs, cut_ys, cut_xs, invalid_mask):
    del cut_zs, cut_ys
    pidx = jnp.arange(P, dtype=jnp.int32)[:, None]
    hidx = jnp.arange(H, dtype=jnp.int32)[None, :]
    xs = cut_xs[pidx * (C * H) + hidx]
    inv = invalid_mask[:, 0, :, 0]
    scode_t = jnp.where(inv, jnp.int32(SENTINEL), xs).T
    feats_flat = features.transpose(1, 0, 2, 3).reshape(-1)
    out = _sc_gather(feats_flat, scode_t.reshape(-1))
    return out.transpose(0, 4, 1, 2, 3)
